# baseline (speedup 1.0000x reference)
import functools
import jax
import jax.numpy as jnp
from jax import lax
from jax.experimental import pallas as pl
from jax.experimental.pallas import tpu as pltpu
from jax.experimental.pallas import tpu_sc as plsc

N = 16384
K = 512
NC, NS, L = 2, 16, 16
NW = NC * NS
CH = 16
KV = K // L


def _xlane(v, idx):
    dnums = lax.GatherDimensionNumbers(
        offset_dims=(), collapsed_slice_dims=(0,), start_index_map=(0,))
    return lax.gather(v, idx[:, None], dnums, (1,),
                      mode=lax.GatherScatterMode.PROMISE_IN_BOUNDS)


def _sc_body(n_rows, row_off, t_hbm, out_hbm, s_v, p_v, o_v, w_v):
    wid = lax.axis_index("s") * NC + lax.axis_index("c")
    rows_per_w = n_rows // NW
    base = row_off + wid * rows_per_w
    lane = lax.iota(jnp.int32, L)

    @pl.loop(0, rows_per_w // CH)
    def _chunk(ci):
        row0 = base + ci * CH
        pltpu.sync_copy(t_hbm.at[0, pl.ds(row0, CH), :], s_v)
        pltpu.sync_copy(t_hbm.at[1, pl.ds(row0, CH), :], p_v)
        pltpu.sync_copy(t_hbm.at[2, pl.ds(row0, CH), :], o_v)
        w = jnp.zeros((L,), jnp.float32)
        for r in range(CH):
            acc = jnp.zeros((L,), jnp.float32)
            for t in range(KV):
                d = (s_v[r, pl.ds(t * L, L)] + p_v[r, pl.ds(t * L, L)]
                     - o_v[r, pl.ds(t * L, L)])
                acc = acc + d * d
            for s in (8, 4, 2, 1):
                acc = acc + _xlane(acc, lane ^ s)
            w = jnp.where(lane == r, acc, w)
        x = jnp.maximum(w, jnp.float32(1e-30))
        i = lax.bitcast_convert_type(x, jnp.int32)
        y = lax.bitcast_convert_type(jnp.int32(0x5F3759DF) - (i >> 1),
                                     jnp.float32)
        half = x * jnp.float32(0.5)
        for _ in range(3):
            y = y * (jnp.float32(1.5) - half * y * y)
        w_v[...] = -(x * y)
        pltpu.sync_copy(w_v, out_hbm.at[pl.ds(row0 - row_off, CH)])


def _make_sc_scorer(n_rows, row_off):
    mesh = plsc.VectorSubcoreMesh(core_axis_name="c", subcore_axis_name="s",
                                  num_cores=NC, num_subcores=NS)
    return pl.kernel(
        functools.partial(_sc_body, n_rows, row_off),
        out_type=jax.ShapeDtypeStruct((n_rows,), jnp.float32),
        mesh=mesh,
        scratch_types=[
            pltpu.VMEM((CH, K), jnp.float32),
            pltpu.VMEM((CH, K), jnp.float32),
            pltpu.VMEM((CH, K), jnp.float32),
            pltpu.VMEM((L,), jnp.float32),
        ],
    )


def kernel(triples):
    return _make_sc_scorer(N, 0)(triples)


# speedup vs baseline: 1.0046x; 1.0046x over previous
import functools
import jax
import jax.numpy as jnp
from jax import lax
from jax.experimental import pallas as pl
from jax.experimental.pallas import tpu as pltpu
from jax.experimental.pallas import tpu_sc as plsc

N = 16384
K = 512
NC, NS, L = 2, 16, 16
NW = NC * NS
CH = 16
KV = K // L


def _xlane(v, idx):
    dnums = lax.GatherDimensionNumbers(
        offset_dims=(), collapsed_slice_dims=(0,), start_index_map=(0,))
    return lax.gather(v, idx[:, None], dnums, (1,),
                      mode=lax.GatherScatterMode.PROMISE_IN_BOUNDS)


def _sc_body(n_rows, row_off, t_hbm, out_hbm, s_v, p_v, o_v, w_v):
    wid = lax.axis_index("s") * NC + lax.axis_index("c")
    rows_per_w = n_rows // NW
    base = row_off + wid * rows_per_w
    lane = lax.iota(jnp.int32, L)

    @pl.loop(0, rows_per_w // CH)
    def _chunk(ci):
        row0 = base + ci * CH
        pltpu.sync_copy(t_hbm.at[0, pl.ds(row0, CH), :], s_v)
        pltpu.sync_copy(t_hbm.at[1, pl.ds(row0, CH), :], p_v)
        pltpu.sync_copy(t_hbm.at[2, pl.ds(row0, CH), :], o_v)
        w = jnp.zeros((L,), jnp.float32)
        for r in range(CH):
            accs = [jnp.zeros((L,), jnp.float32) for _ in range(4)]
            for t in range(KV):
                d = (s_v[r, pl.ds(t * L, L)] + p_v[r, pl.ds(t * L, L)]
                     - o_v[r, pl.ds(t * L, L)])
                accs[t % 4] = accs[t % 4] + d * d
            acc = (accs[0] + accs[1]) + (accs[2] + accs[3])
            for s in (8, 4, 2, 1):
                acc = acc + _xlane(acc, lane ^ s)
            w = jnp.where(lane == r, acc, w)
        x = jnp.maximum(w, jnp.float32(1e-30))
        i = lax.bitcast_convert_type(x, jnp.int32)
        y = lax.bitcast_convert_type(jnp.int32(0x5F3759DF) - (i >> 1),
                                     jnp.float32)
        half = x * jnp.float32(0.5)
        for _ in range(3):
            y = y * (jnp.float32(1.5) - half * y * y)
        w_v[...] = -(x * y)
        pltpu.sync_copy(w_v, out_hbm.at[pl.ds(row0 - row_off, CH)])


def _make_sc_scorer(n_rows, row_off):
    mesh = plsc.VectorSubcoreMesh(core_axis_name="c", subcore_axis_name="s",
                                  num_cores=NC, num_subcores=NS)
    return pl.kernel(
        functools.partial(_sc_body, n_rows, row_off),
        out_type=jax.ShapeDtypeStruct((n_rows,), jnp.float32),
        mesh=mesh,
        scratch_types=[
            pltpu.VMEM((CH, K), jnp.float32),
            pltpu.VMEM((CH, K), jnp.float32),
            pltpu.VMEM((CH, K), jnp.float32),
            pltpu.VMEM((L,), jnp.float32),
        ],
    )


def kernel(triples):
    return _make_sc_scorer(N, 0)(triples)


# speedup vs baseline: 1.0535x; 1.0486x over previous
import functools
import jax
import jax.numpy as jnp
from jax import lax
from jax.experimental import pallas as pl
from jax.experimental.pallas import tpu as pltpu
from jax.experimental.pallas import tpu_sc as plsc

N = 16384
K = 512
NC, NS, L = 2, 16, 16
NW = NC * NS
CH = 32
KV = K // L


def _xlane(v, idx):
    dnums = lax.GatherDimensionNumbers(
        offset_dims=(), collapsed_slice_dims=(0,), start_index_map=(0,))
    return lax.gather(v, idx[:, None], dnums, (1,),
                      mode=lax.GatherScatterMode.PROMISE_IN_BOUNDS)


def _sc_body(n_rows, row_off, t_hbm, out_hbm, s_v, p_v, o_v, w_v, sem):
    wid = lax.axis_index("s") * NC + lax.axis_index("c")
    rows_per_w = n_rows // NW
    base = row_off + wid * rows_per_w
    lane = lax.iota(jnp.int32, L)

    @pl.loop(0, rows_per_w // CH)
    def _chunk(ci):
        row0 = base + ci * CH
        pltpu.async_copy(t_hbm.at[0, pl.ds(row0, CH), :], s_v, sem)
        pltpu.async_copy(t_hbm.at[1, pl.ds(row0, CH), :], p_v, sem)
        c3 = pltpu.async_copy(t_hbm.at[2, pl.ds(row0, CH), :], o_v, sem)
        pltpu.make_async_copy(t_hbm.at[0, pl.ds(row0, CH), :], s_v, sem).wait()
        pltpu.make_async_copy(t_hbm.at[1, pl.ds(row0, CH), :], p_v, sem).wait()
        c3.wait()
        for g in range(CH // L):
            w = jnp.zeros((L,), jnp.float32)
            for rr in range(L):
                r = g * L + rr
                accs = [jnp.zeros((L,), jnp.float32) for _ in range(4)]
                for t in range(KV):
                    d = (s_v[r, pl.ds(t * L, L)] + p_v[r, pl.ds(t * L, L)]
                         - o_v[r, pl.ds(t * L, L)])
                    accs[t % 4] = accs[t % 4] + d * d
                acc = (accs[0] + accs[1]) + (accs[2] + accs[3])
                for s in (8, 4, 2, 1):
                    acc = acc + _xlane(acc, lane ^ s)
                w = jnp.where(lane == rr, acc, w)
            x = jnp.maximum(w, jnp.float32(1e-30))
            i = lax.bitcast_convert_type(x, jnp.int32)
            y = lax.bitcast_convert_type(jnp.int32(0x5F3759DF) - (i >> 1),
                                         jnp.float32)
            half = x * jnp.float32(0.5)
            for _ in range(3):
                y = y * (jnp.float32(1.5) - half * y * y)
            w_v[pl.ds(g * L, L)] = -(x * y)
        pltpu.sync_copy(w_v, out_hbm.at[pl.ds(row0 - row_off, CH)])


def _make_sc_scorer(n_rows, row_off):
    mesh = plsc.VectorSubcoreMesh(core_axis_name="c", subcore_axis_name="s",
                                  num_cores=NC, num_subcores=NS)
    return pl.kernel(
        functools.partial(_sc_body, n_rows, row_off),
        out_type=jax.ShapeDtypeStruct((n_rows,), jnp.float32),
        mesh=mesh,
        scratch_types=[
            pltpu.VMEM((CH, K), jnp.float32),
            pltpu.VMEM((CH, K), jnp.float32),
            pltpu.VMEM((CH, K), jnp.float32),
            pltpu.VMEM((CH,), jnp.float32),
            pltpu.SemaphoreType.DMA,
        ],
    )


def kernel(triples):
    return _make_sc_scorer(N, 0)(triples)


# speedup vs baseline: 4.4288x; 4.2038x over previous
import functools
import jax
import jax.numpy as jnp
from jax import lax
from jax.experimental import pallas as pl
from jax.experimental.pallas import tpu as pltpu
from jax.experimental.pallas import tpu_sc as plsc

N = 16384
K = 512
NC, NS, L = 2, 16, 16
NW = NC * NS
CH = 32
KV = K // L


def _xlane(v, idx):
    dnums = lax.GatherDimensionNumbers(
        offset_dims=(), collapsed_slice_dims=(0,), start_index_map=(0,))
    return lax.gather(v, idx[:, None], dnums, (1,),
                      mode=lax.GatherScatterMode.PROMISE_IN_BOUNDS)


def _sc_body(n_rows, row_off, t_hbm, out_hbm, s_v, p_v, o_v, w_v, sem):
    wid = lax.axis_index("s") * NC + lax.axis_index("c")
    rows_per_w = n_rows // NW
    base = row_off + wid * rows_per_w
    lane = lax.iota(jnp.int32, L)

    @pl.loop(0, rows_per_w // CH)
    def _chunk(ci):
        row0 = base + ci * CH
        pltpu.async_copy(t_hbm.at[0, pl.ds(row0, CH), :], s_v, sem)
        pltpu.async_copy(t_hbm.at[1, pl.ds(row0, CH), :], p_v, sem)
        c3 = pltpu.async_copy(t_hbm.at[2, pl.ds(row0, CH), :], o_v, sem)
        pltpu.make_async_copy(t_hbm.at[0, pl.ds(row0, CH), :], s_v, sem).wait()
        pltpu.make_async_copy(t_hbm.at[1, pl.ds(row0, CH), :], p_v, sem).wait()
        c3.wait()
        for g in range(CH // L):
            w = jnp.zeros((L,), jnp.float32)
            for rr in range(L):
                r = g * L + rr
                accs = [jnp.zeros((L,), jnp.float32) for _ in range(4)]
                for t in range(KV):
                    d = (s_v[r, pl.ds(t * L, L)] + p_v[r, pl.ds(t * L, L)]
                         - o_v[r, pl.ds(t * L, L)])
                    accs[t % 4] = accs[t % 4] + d * d
                acc = (accs[0] + accs[1]) + (accs[2] + accs[3])
                for s in (8, 4, 2, 1):
                    acc = acc + _xlane(acc, lane ^ s)
                w = jnp.where(lane == rr, acc, w)
            x = jnp.maximum(w, jnp.float32(1e-30))
            i = lax.bitcast_convert_type(x, jnp.int32)
            y = lax.bitcast_convert_type(jnp.int32(0x5F3759DF) - (i >> 1),
                                         jnp.float32)
            half = x * jnp.float32(0.5)
            for _ in range(3):
                y = y * (jnp.float32(1.5) - half * y * y)
            w_v[pl.ds(g * L, L)] = -(x * y)
        pltpu.sync_copy(w_v, out_hbm.at[pl.ds(row0 - row_off, CH)])


def _make_sc_scorer(n_rows, row_off):
    mesh = plsc.VectorSubcoreMesh(core_axis_name="c", subcore_axis_name="s",
                                  num_cores=NC, num_subcores=NS)
    return pl.kernel(
        functools.partial(_sc_body, n_rows, row_off),
        out_type=jax.ShapeDtypeStruct((n_rows,), jnp.float32),
        mesh=mesh,
        scratch_types=[
            pltpu.VMEM((CH, K), jnp.float32),
            pltpu.VMEM((CH, K), jnp.float32),
            pltpu.VMEM((CH, K), jnp.float32),
            pltpu.VMEM((CH,), jnp.float32),
            pltpu.SemaphoreType.DMA,
        ],
    )


N_SC = 2048
N_TC = N - N_SC
BN = 2048


def _tc_block(t_ref, o_ref):
    d = t_ref[0] + t_ref[1] - t_ref[2]
    o_ref[...] = -jnp.sqrt(jnp.sum(d * d, axis=1))


def _tc_score(triples):
    return pl.pallas_call(
        _tc_block,
        grid=(N_TC // BN,),
        in_specs=[pl.BlockSpec((3, BN, K), lambda i: (0, i, 0))],
        out_specs=pl.BlockSpec((BN,), lambda i: (i,)),
        out_shape=jax.ShapeDtypeStruct((N_TC,), jnp.float32),
    )(triples)


def kernel(triples):
    sc_scores = _make_sc_scorer(N_SC, N_TC)(triples)
    tc_scores = _tc_score(triples)
    return jnp.concatenate([tc_scores, sc_scores])


# speedup vs baseline: 4.4476x; 1.0042x over previous
import functools
import jax
import jax.numpy as jnp
from jax import lax
from jax.experimental import pallas as pl
from jax.experimental.pallas import tpu as pltpu
from jax.experimental.pallas import tpu_sc as plsc

N = 16384
K = 512
NC, NS, L = 2, 16, 16
NW = NC * NS
CH = 32
KV = K // L


def _xlane(v, idx):
    dnums = lax.GatherDimensionNumbers(
        offset_dims=(), collapsed_slice_dims=(0,), start_index_map=(0,))
    return lax.gather(v, idx[:, None], dnums, (1,),
                      mode=lax.GatherScatterMode.PROMISE_IN_BOUNDS)


def _sc_body(n_rows, row_off, t_hbm, out_hbm, s_v, p_v, o_v, w_v, sem):
    wid = lax.axis_index("s") * NC + lax.axis_index("c")
    rows_per_w = n_rows // NW
    base = row_off + wid * rows_per_w
    lane = lax.iota(jnp.int32, L)

    @pl.loop(0, rows_per_w // CH)
    def _chunk(ci):
        row0 = base + ci * CH
        pltpu.async_copy(t_hbm.at[0, pl.ds(row0, CH), :], s_v, sem)
        pltpu.async_copy(t_hbm.at[1, pl.ds(row0, CH), :], p_v, sem)
        c3 = pltpu.async_copy(t_hbm.at[2, pl.ds(row0, CH), :], o_v, sem)
        pltpu.make_async_copy(t_hbm.at[0, pl.ds(row0, CH), :], s_v, sem).wait()
        pltpu.make_async_copy(t_hbm.at[1, pl.ds(row0, CH), :], p_v, sem).wait()
        c3.wait()
        for g in range(CH // L):
            w = jnp.zeros((L,), jnp.float32)
            for rr in range(L):
                r = g * L + rr
                accs = [jnp.zeros((L,), jnp.float32) for _ in range(4)]
                for t in range(KV):
                    d = (s_v[r, pl.ds(t * L, L)] + p_v[r, pl.ds(t * L, L)]
                         - o_v[r, pl.ds(t * L, L)])
                    accs[t % 4] = accs[t % 4] + d * d
                acc = (accs[0] + accs[1]) + (accs[2] + accs[3])
                for s in (8, 4, 2, 1):
                    acc = acc + _xlane(acc, lane ^ s)
                w = jnp.where(lane == rr, acc, w)
            x = jnp.maximum(w, jnp.float32(1e-30))
            i = lax.bitcast_convert_type(x, jnp.int32)
            y = lax.bitcast_convert_type(jnp.int32(0x5F3759DF) - (i >> 1),
                                         jnp.float32)
            half = x * jnp.float32(0.5)
            for _ in range(3):
                y = y * (jnp.float32(1.5) - half * y * y)
            w_v[pl.ds(g * L, L)] = -(x * y)
        pltpu.sync_copy(w_v, out_hbm.at[pl.ds(row0 - row_off, CH)])


def _make_sc_scorer(n_rows, row_off):
    mesh = plsc.VectorSubcoreMesh(core_axis_name="c", subcore_axis_name="s",
                                  num_cores=NC, num_subcores=NS)
    return pl.kernel(
        functools.partial(_sc_body, n_rows, row_off),
        out_type=jax.ShapeDtypeStruct((n_rows,), jnp.float32),
        mesh=mesh,
        scratch_types=[
            pltpu.VMEM((CH, K), jnp.float32),
            pltpu.VMEM((CH, K), jnp.float32),
            pltpu.VMEM((CH, K), jnp.float32),
            pltpu.VMEM((CH,), jnp.float32),
            pltpu.SemaphoreType.DMA,
        ],
    )


N_SC = 2048
N_TC = N - N_SC
BN = 2048


def _tc_block(t_ref, o_ref):
    d = t_ref[0] + t_ref[1] - t_ref[2]
    o_ref[...] = -jnp.sqrt(jnp.sum(d * d, axis=1))


def _tc_score(triples):
    return pl.pallas_call(
        _tc_block,
        grid=(N_TC // BN,),
        in_specs=[pl.BlockSpec((3, BN, K), lambda i: (0, i, 0))],
        out_specs=pl.BlockSpec((BN,), lambda i: (i,)),
        out_shape=jax.ShapeDtypeStruct((N_TC,), jnp.float32),
    )(triples)


def kernel(triples):
    tc_scores = _tc_score(triples)
    sc_scores = _make_sc_scorer(N_SC, N_TC)(triples)
    return jnp.concatenate([tc_scores, sc_scores])


# speedup vs baseline: 8.4211x; 1.8934x over previous
import jax
import jax.numpy as jnp
from jax.experimental import pallas as pl

N = 16384
K = 512
BN = 2048


def _score_block(t_ref, o_ref):
    d = t_ref[0] + t_ref[1] - t_ref[2]
    o_ref[...] = -jnp.sqrt(jnp.sum(d * d, axis=1))


def kernel(triples):
    return pl.pallas_call(
        _score_block,
        grid=(N // BN,),
        in_specs=[pl.BlockSpec((3, BN, K), lambda i: (0, i, 0))],
        out_specs=pl.BlockSpec((BN,), lambda i: (i,)),
        out_shape=jax.ShapeDtypeStruct((N,), jnp.float32),
    )(triples)
